# Initial kernel scaffold; baseline (speedup 1.0000x reference)
import jax
import jax.numpy as jnp
from jax.experimental import pallas as pl


def kernel(x, target_vec, table, W, b):
    raise NotImplementedError("write your pallas kernel here")


# speedup vs baseline: 2.9577x; 2.9577x over previous
import functools
import math

import jax
import jax.numpy as jnp
from jax import lax
from jax.experimental import pallas as pl
from jax.experimental.pallas import tpu as pltpu
from jax.experimental.pallas import tpu_sc as plsc

D_MODEL = 128
SCALE = math.sqrt(128.0)

NC, NS = 2, 16
NW = NC * NS
C = 128
NCHUNK = 50
NB = 5
ROWS_PER_W = C * NCHUNK
B_TOTAL = NW * ROWS_PER_W


def _scale_chunk(buf):
    def row(i, carry):
        for h in range(D_MODEL // 16):
            sl = (i, pl.ds(h * 16, 16))
            buf[sl] = buf[sl] * SCALE
        return carry
    lax.fori_loop(0, C, row, 0, unroll=2)


def _sc_body(x_hbm, table_hbm, out_hbm, idx_v, rows, *sems):
    gs = sems[:NB]
    ss = sems[NB:]
    wid = lax.axis_index("s") * NC + lax.axis_index("c")
    base = wid * ROWS_PER_W

    pltpu.sync_copy(x_hbm.at[wid], idx_v)

    def gather(j, b):
        return pltpu.make_async_copy(
            table_hbm.at[idx_v.at[j]], rows.at[b], gs[b])

    def scatter(j, b):
        return pltpu.make_async_copy(
            rows.at[b], out_hbm.at[pl.ds(base + j * C, C)], ss[b])

    for b in range(NB):
        gather(b, b).start()

    def outer(g, carry):
        for b in range(NB):
            j = g * NB + b
            k = j + NB - 2
            bk = (b + NB - 2) % NB

            @pl.when(jnp.logical_and(k >= NB, k < NCHUNK))
            def _():
                scatter(k - NB, bk).wait()
                gather(k, bk).start()

            gather(j, b).wait()
            _scale_chunk(rows.at[b])
            scatter(j, b).start()
        return carry

    lax.fori_loop(0, NCHUNK // NB, outer, 0)

    for b in range(NB):
        scatter(0, b).wait()


@functools.partial(jax.jit, static_argnums=())
def _embed_scaled(x_flat, table):
    k = pl.kernel(
        _sc_body,
        out_type=jax.ShapeDtypeStruct((B_TOTAL, D_MODEL), jnp.float32),
        mesh=plsc.VectorSubcoreMesh(core_axis_name="c", subcore_axis_name="s"),
        scratch_types=(
            [pltpu.VMEM((NCHUNK, C), jnp.int32),
             pltpu.VMEM((NB, C, D_MODEL), jnp.float32)]
            + [pltpu.SemaphoreType.DMA] * (2 * NB)
        ),
    )
    return k(x_flat, table)


def kernel(x, target_vec, table, W, b):
    bsz, seq = x.shape
    x_flat = x.astype(jnp.int32).reshape(NW, NCHUNK, C)
    out = _embed_scaled(x_flat, table)
    return out.reshape(bsz, seq, D_MODEL)


# speedup vs baseline: 5.2598x; 1.7784x over previous
import functools
import math

import jax
import jax.numpy as jnp
from jax import lax
from jax.experimental import pallas as pl
from jax.experimental.pallas import tpu as pltpu
from jax.experimental.pallas import tpu_sc as plsc

D_MODEL = 128
SEQ = 50
SCALE = math.sqrt(128.0)

NC, NS = 2, 16
NW = NC * NS
BPC = 2
NCHUNK = 64
BATCH_PER_W = BPC * NCHUNK
NB = 4
BATCH = NW * BATCH_PER_W


def _scale_chunk(buf):
    def row(i, carry):
        for p in range(BPC):
            for h in range(D_MODEL // 16):
                sl = (p, i, pl.ds(h * 16, 16))
                buf[sl] = buf[sl] * SCALE
        return carry
    lax.fori_loop(0, SEQ, row, 0, unroll=2)


def _sc_body(x_hbm, table_hbm, out_hbm, idx_v, rows, *sems):
    gs = sems[:2 * NB]
    ss = sems[2 * NB:]
    wid = lax.axis_index("s") * NC + lax.axis_index("c")
    base = wid * BATCH_PER_W

    pltpu.sync_copy(x_hbm.at[wid], idx_v)

    def gather(j, b, p):
        return pltpu.make_async_copy(
            table_hbm.at[idx_v.at[j, p]], rows.at[b, p], gs[2 * b + p])

    def scatter(j, b):
        return pltpu.make_async_copy(
            rows.at[b], out_hbm.at[pl.ds(base + j * BPC, BPC)], ss[b])

    for b in range(NB):
        for p in range(BPC):
            gather(b, b, p).start()

    def outer(g, carry):
        for b in range(NB):
            j = g * NB + b
            k = j + NB - 2
            bk = (b + NB - 2) % NB

            @pl.when(jnp.logical_and(k >= NB, k < NCHUNK))
            def _():
                scatter(k - NB, bk).wait()
                for p in range(BPC):
                    gather(k, bk, p).start()

            for p in range(BPC):
                gather(j, b, p).wait()
            _scale_chunk(rows.at[b])
            scatter(j, b).start()
        return carry

    lax.fori_loop(0, NCHUNK // NB, outer, 0)

    for b in range(NB):
        scatter(0, b).wait()


@jax.jit
def _embed_scaled(x_flat, table):
    k = pl.kernel(
        _sc_body,
        out_type=jax.ShapeDtypeStruct((BATCH, SEQ, D_MODEL), jnp.float32),
        mesh=plsc.VectorSubcoreMesh(core_axis_name="c", subcore_axis_name="s"),
        scratch_types=(
            [pltpu.VMEM((NCHUNK, BPC, SEQ), jnp.int32),
             pltpu.VMEM((NB, BPC, SEQ, D_MODEL), jnp.float32)]
            + [pltpu.SemaphoreType.DMA] * (2 * NB + NB)
        ),
    )
    return k(x_flat, table)


def kernel(x, target_vec, table, W, b):
    bsz, seq = x.shape
    x_flat = x.astype(jnp.int32).reshape(NW, NCHUNK, BPC, seq)
    return _embed_scaled(x_flat, table)


# speedup vs baseline: 9.4176x; 1.7905x over previous
import math

import jax
import jax.numpy as jnp
from jax import lax
from jax.experimental import pallas as pl
from jax.experimental.pallas import tpu as pltpu
from jax.experimental.pallas import tpu_sc as plsc

D_MODEL = 128
SEQ = 50
SCALE = math.sqrt(128.0)

NC, NS = 2, 16
NW = NC * NS
C = 128
NCHUNK = SEQ
NB = 5
BATCH = NW * C


def _scale_chunk(buf):
    def row(i, carry):
        for h in range(D_MODEL // 16):
            sl = (i, pl.ds(h * 16, 16))
            buf[sl] = buf[sl] * SCALE
        return carry
    lax.fori_loop(0, C, row, 0, unroll=2)


def _sc_body(x_hbm, table_hbm, out_hbm, idx_v, rows, *sems):
    gs = sems[:NB]
    ss = sems[NB:]
    wid = lax.axis_index("s") * NC + lax.axis_index("c")
    base = wid * C

    pltpu.sync_copy(x_hbm.at[wid], idx_v)

    def gather(j, b):
        return pltpu.make_async_copy(
            table_hbm.at[idx_v.at[j]], rows.at[b], gs[b])

    def scatter(j, b):
        return pltpu.make_async_copy(
            rows.at[b], out_hbm.at[j, pl.ds(base, C)], ss[b])

    for b in range(NB):
        gather(b, b).start()

    def outer(g, carry):
        for b in range(NB):
            j = g * NB + b
            k = j + NB - 2
            bk = (b + NB - 2) % NB

            @pl.when(jnp.logical_and(k >= NB, k < NCHUNK))
            def _():
                scatter(k - NB, bk).wait()
                gather(k, bk).start()

            gather(j, b).wait()
            _scale_chunk(rows.at[b])
            scatter(j, b).start()
        return carry

    lax.fori_loop(0, NCHUNK // NB, outer, 0)

    for b in range(NB):
        scatter(0, b).wait()


@jax.jit
def _embed_scaled(x_w, table):
    k = pl.kernel(
        _sc_body,
        out_type=jax.ShapeDtypeStruct((SEQ, BATCH, D_MODEL), jnp.float32),
        mesh=plsc.VectorSubcoreMesh(core_axis_name="c", subcore_axis_name="s"),
        scratch_types=(
            [pltpu.VMEM((NCHUNK, C), jnp.int32),
             pltpu.VMEM((NB, C, D_MODEL), jnp.float32)]
            + [pltpu.SemaphoreType.DMA] * (2 * NB)
        ),
    )
    return k(x_w, table)


def kernel(x, target_vec, table, W, b):
    bsz, seq = x.shape
    x_w = jnp.transpose(
        x.astype(jnp.int32).T.reshape(seq, NW, C), (1, 0, 2))
    out_sm = _embed_scaled(x_w, table)
    return jnp.transpose(out_sm, (1, 0, 2))
